# baseline (speedup 1.0000x reference)
import dataclasses
import functools

import jax
import jax.numpy as jnp
from jax import lax
from jax.experimental import pallas as pl
from jax.experimental.pallas import tpu as pltpu
from jax.experimental.pallas import tpu_sc as plsc

N_NODES = 10000
N_PAD = 10240
N_EDGES = 320000
D_FEAT = 128

NC = 2
NS = 16
NW = NC * NS
EPW = N_EDGES // NW
CHUNK = 80
NCHUNK = EPW // CHUNK
ROWS_PER_SUB = N_PAD // NS

_f32 = jnp.float32


def _sc_agg():
    mesh = plsc.VectorSubcoreMesh(core_axis_name="c", subcore_axis_name="s")
    out_type = jax.ShapeDtypeStruct((NC * N_PAD, D_FEAT), _f32)
    scratch = [
        pltpu.VMEM((EPW,), jnp.int32),
        pltpu.VMEM((NCHUNK, CHUNK), jnp.int32),
        pltpu.VMEM((CHUNK, D_FEAT), _f32),
        pltpu.VMEM((CHUNK, D_FEAT), _f32),
        pltpu.VMEM_SHARED((N_PAD, D_FEAT), _f32),
        pltpu.SemaphoreType.DMA,
        pltpu.SemaphoreType.DMA,
    ]

    @functools.partial(pl.kernel, out_type=out_type, mesh=mesh,
                       scratch_types=scratch)
    def agg(x_hbm, src_hbm, dst3_hbm, z_hbm, out_hbm, src_v, dst_v, rows_a,
            rows_b, acc_sh, sem_a, sem_b):
        c = lax.axis_index("c")
        s = lax.axis_index("s")
        wid = s * NC + c
        base_row = s * ROWS_PER_SUB

        stage_src = pltpu.async_copy(src_hbm.at[pl.ds(wid * EPW, EPW)],
                                     src_v, sem_a)
        stage_dst = pltpu.async_copy(dst3_hbm.at[wid], dst_v, sem_b)
        pltpu.sync_copy(z_hbm, acc_sh.at[pl.ds(base_row, ROWS_PER_SUB)])

        stage_src.wait()
        stage_dst.wait()
        plsc.subcore_barrier()

        def gather(i, buf, sem):
            return pltpu.async_copy(
                x_hbm.at[src_v.at[pl.ds(i * CHUNK, CHUNK)]], buf, sem)

        def finish(i, buf, sem):
            pltpu.make_async_copy(
                x_hbm.at[src_v.at[pl.ds(i * CHUNK, CHUNK)]], buf, sem).wait()
            pltpu.sync_copy(buf, acc_sh.at[dst_v.at[i]], add=True)

        gather(0, rows_a, sem_a)

        @pl.loop(0, NCHUNK - 1, step=2)
        def _(i):
            gather(i + 1, rows_b, sem_b)
            finish(i, rows_a, sem_a)
            gather(i + 2, rows_a, sem_a)
            finish(i + 1, rows_b, sem_b)

        finish(NCHUNK - 1, rows_a, sem_a)

        plsc.subcore_barrier()

        out_base = c * N_PAD + base_row
        pltpu.sync_copy(acc_sh.at[pl.ds(base_row, ROWS_PER_SUB)],
                        out_hbm.at[pl.ds(out_base, ROWS_PER_SUB)])

    return agg


def _sc_counts():
    mesh = plsc.VectorSubcoreMesh(core_axis_name="c", subcore_axis_name="s")
    out_type = jax.ShapeDtypeStruct((NC * NS * N_PAD,), _f32)
    scratch = [
        pltpu.VMEM((EPW,), jnp.int32),
        pltpu.VMEM((N_PAD,), _f32),
    ]
    cp = pltpu.CompilerParams()
    if "needs_layout_passes" in pltpu.CompilerParams.__dataclass_fields__:
        cp = dataclasses.replace(cp, needs_layout_passes=False)

    @functools.partial(pl.kernel, out_type=out_type, mesh=mesh,
                       scratch_types=scratch, compiler_params=cp)
    def cntk(dst_hbm, out_hbm, dst_v, cnt_v):
        c = lax.axis_index("c")
        s = lax.axis_index("s")
        wid = s * NC + c

        zero16 = jnp.zeros((16,), _f32)
        ones16 = jnp.full((16,), 1.0, _f32)

        pltpu.sync_copy(dst_hbm.at[pl.ds(wid * EPW, EPW)], dst_v)

        @pl.loop(0, N_PAD, step=16)
        def _(k):
            cnt_v[pl.ds(k, 16)] = zero16

        @pl.loop(0, EPW, step=16)
        def _(j0):
            d16 = dst_v[pl.ds(j0, 16)]
            plsc.addupdate_scatter(cnt_v, [d16], ones16)

        pltpu.sync_copy(cnt_v, out_hbm.at[pl.ds(wid * N_PAD, N_PAD)])

    return cntk


_sc_agg_k = _sc_agg()
_sc_counts_k = _sc_counts()

BR = 1024


def _tc_matmul(x, w, b):
    dout = w.shape[1]

    def body(x_ref, w_ref, b_ref, o_ref):
        o_ref[...] = jnp.dot(x_ref[...], w_ref[...],
                             preferred_element_type=_f32) + b_ref[...]

    return pl.pallas_call(
        body,
        grid=(pl.cdiv(N_NODES, BR),),
        in_specs=[
            pl.BlockSpec((BR, x.shape[1]), lambda i: (i, 0)),
            pl.BlockSpec((x.shape[1], dout), lambda i: (0, 0)),
            pl.BlockSpec((1, dout), lambda i: (0, 0)),
        ],
        out_specs=pl.BlockSpec((BR, dout), lambda i: (i, 0)),
        out_shape=jax.ShapeDtypeStruct((N_NODES, dout), _f32),
    )(x, w, b.reshape(1, dout))


def _tc_combine(xs, acc, cnt, w_neigh, act):
    dout = w_neigh.shape[1]

    def body(xs_ref, acc_ref, cnt_ref, w_ref, o_ref):
        total = acc_ref[0] + acc_ref[1]
        n = jnp.sum(cnt_ref[...], axis=0)
        mean = total * (1.0 / jnp.maximum(n, 1.0))[:, None]
        r = xs_ref[...] + jnp.dot(mean, w_ref[...],
                                  preferred_element_type=_f32)
        if act == "relu":
            o_ref[...] = jnp.maximum(r, 0.0)
        else:
            m = jnp.max(r, axis=1, keepdims=True)
            e = jnp.exp(r - m)
            o_ref[...] = e / jnp.sum(e, axis=1, keepdims=True)

    return pl.pallas_call(
        body,
        grid=(pl.cdiv(N_NODES, BR),),
        in_specs=[
            pl.BlockSpec((BR, dout), lambda i: (i, 0)),
            pl.BlockSpec((NC, BR, D_FEAT), lambda i: (0, i, 0)),
            pl.BlockSpec((NC * NS, BR), lambda i: (0, i)),
            pl.BlockSpec((D_FEAT, dout), lambda i: (0, 0)),
        ],
        out_specs=pl.BlockSpec((BR, dout), lambda i: (i, 0)),
        out_shape=jax.ShapeDtypeStruct((N_NODES, dout), _f32),
    )(xs, acc, cnt, w_neigh)


def kernel(x, edge_index, W1_self, W1_neigh, b1, W2_self, W2_neigh, b2):
    src = edge_index[0].astype(jnp.int32)
    dst = edge_index[1].astype(jnp.int32)
    dst3 = dst.reshape(NW, NCHUNK, CHUNK)
    zeros = jnp.zeros((ROWS_PER_SUB, D_FEAT), _f32)

    acc1 = _sc_agg_k(x, src, dst3, zeros).reshape(NC, N_PAD, D_FEAT)
    cnt = _sc_counts_k(dst).reshape(NC * NS, N_PAD)
    xs = _tc_matmul(x, W1_self, b1)
    h = _tc_combine(xs, acc1, cnt, W1_neigh, "relu")

    acc2 = _sc_agg_k(h, src, dst3, zeros).reshape(NC, N_PAD, D_FEAT)
    hs = _tc_matmul(h, W2_self, b2)
    return _tc_combine(hs, acc2, cnt, W2_neigh, "softmax")


# speedup vs baseline: 1.0395x; 1.0395x over previous
import dataclasses
import functools

import jax
import jax.numpy as jnp
from jax import lax
from jax.experimental import pallas as pl
from jax.experimental.pallas import tpu as pltpu
from jax.experimental.pallas import tpu_sc as plsc

N_NODES = 10000
N_PAD = 10240
N_EDGES = 320000
D_FEAT = 128

NC = 2
NS = 16
NW = NC * NS
EPW = N_EDGES // NW
CHUNK = 80
NCHUNK = EPW // CHUNK
ROWS_PER_SUB = N_PAD // NS

_f32 = jnp.float32


def _sc_agg():
    mesh = plsc.VectorSubcoreMesh(core_axis_name="c", subcore_axis_name="s")
    out_type = jax.ShapeDtypeStruct((NC * N_PAD, D_FEAT), _f32)
    scratch = [
        pltpu.VMEM((EPW,), jnp.int32),
        pltpu.VMEM((NCHUNK, CHUNK), jnp.int32),
        pltpu.VMEM((CHUNK, D_FEAT), _f32),
        pltpu.VMEM((CHUNK, D_FEAT), _f32),
        pltpu.VMEM_SHARED((N_PAD, D_FEAT), _f32),
        pltpu.SemaphoreType.DMA,
        pltpu.SemaphoreType.DMA,
    ]

    @functools.partial(pl.kernel, out_type=out_type, mesh=mesh,
                       scratch_types=scratch)
    def agg(x_hbm, src_hbm, dst3_hbm, out_hbm, src_v, dst_v, rows_a, rows_b,
            acc_sh, sem_a, sem_b):
        c = lax.axis_index("c")
        s = lax.axis_index("s")
        wid = s * NC + c
        base_row = s * ROWS_PER_SUB

        zero16 = jnp.zeros((16,), _f32)

        stage_src = pltpu.async_copy(src_hbm.at[pl.ds(wid * EPW, EPW)],
                                     src_v, sem_a)
        stage_dst = pltpu.async_copy(dst3_hbm.at[wid], dst_v, sem_b)

        @pl.loop(0, CHUNK)
        def _(r):
            @pl.loop(0, D_FEAT, step=16)
            def _(col):
                rows_a[r, pl.ds(col, 16)] = zero16

        @pl.loop(0, ROWS_PER_SUB, step=CHUNK)
        def _(r0):
            pltpu.sync_copy(rows_a, acc_sh.at[pl.ds(base_row + r0, CHUNK)])

        stage_src.wait()
        stage_dst.wait()
        plsc.subcore_barrier()

        def gather(i, buf, sem):
            return pltpu.async_copy(
                x_hbm.at[src_v.at[pl.ds(i * CHUNK, CHUNK)]], buf, sem)

        def finish(i, buf, sem):
            pltpu.make_async_copy(
                x_hbm.at[src_v.at[pl.ds(i * CHUNK, CHUNK)]], buf, sem).wait()
            pltpu.sync_copy(buf, acc_sh.at[dst_v.at[i]], add=True)

        gather(0, rows_a, sem_a)

        @pl.loop(0, NCHUNK - 1, step=2)
        def _(i):
            gather(i + 1, rows_b, sem_b)
            finish(i, rows_a, sem_a)
            gather(i + 2, rows_a, sem_a)
            finish(i + 1, rows_b, sem_b)

        finish(NCHUNK - 1, rows_a, sem_a)

        plsc.subcore_barrier()

        out_base = c * N_PAD + base_row
        pltpu.sync_copy(acc_sh.at[pl.ds(base_row, ROWS_PER_SUB)],
                        out_hbm.at[pl.ds(out_base, ROWS_PER_SUB)])

    return agg


def _sc_counts():
    mesh = plsc.VectorSubcoreMesh(core_axis_name="c", subcore_axis_name="s")
    out_type = jax.ShapeDtypeStruct((NC * NS * N_PAD,), _f32)
    scratch = [
        pltpu.VMEM((EPW,), jnp.int32),
        pltpu.VMEM((N_PAD,), _f32),
    ]
    cp = pltpu.CompilerParams()
    if "needs_layout_passes" in pltpu.CompilerParams.__dataclass_fields__:
        cp = dataclasses.replace(cp, needs_layout_passes=False)

    @functools.partial(pl.kernel, out_type=out_type, mesh=mesh,
                       scratch_types=scratch, compiler_params=cp)
    def cntk(dst_hbm, out_hbm, dst_v, cnt_v):
        c = lax.axis_index("c")
        s = lax.axis_index("s")
        wid = s * NC + c

        zero16 = jnp.zeros((16,), _f32)
        ones16 = jnp.full((16,), 1.0, _f32)

        pltpu.sync_copy(dst_hbm.at[pl.ds(wid * EPW, EPW)], dst_v)

        @pl.loop(0, N_PAD, step=16)
        def _(k):
            cnt_v[pl.ds(k, 16)] = zero16

        @pl.loop(0, EPW, step=16)
        def _(j0):
            d16 = dst_v[pl.ds(j0, 16)]
            plsc.addupdate_scatter(cnt_v, [d16], ones16)

        pltpu.sync_copy(cnt_v, out_hbm.at[pl.ds(wid * N_PAD, N_PAD)])

    return cntk


_sc_agg_k = _sc_agg()
_sc_counts_k = _sc_counts()

BR = 1024


def _tc_matmul(x, w, b):
    dout = w.shape[1]

    def body(x_ref, w_ref, b_ref, o_ref):
        o_ref[...] = jnp.dot(x_ref[...], w_ref[...],
                             preferred_element_type=_f32) + b_ref[...]

    return pl.pallas_call(
        body,
        grid=(pl.cdiv(N_NODES, BR),),
        in_specs=[
            pl.BlockSpec((BR, x.shape[1]), lambda i: (i, 0)),
            pl.BlockSpec((x.shape[1], dout), lambda i: (0, 0)),
            pl.BlockSpec((1, dout), lambda i: (0, 0)),
        ],
        out_specs=pl.BlockSpec((BR, dout), lambda i: (i, 0)),
        out_shape=jax.ShapeDtypeStruct((N_NODES, dout), _f32),
    )(x, w, b.reshape(1, dout))


def _tc_combine(xs, acc, cnt, w_neigh, act):
    dout = w_neigh.shape[1]

    def body(xs_ref, acc_ref, cnt_ref, w_ref, o_ref):
        total = acc_ref[0] + acc_ref[1]
        n = jnp.sum(cnt_ref[...], axis=0)
        mean = total * (1.0 / jnp.maximum(n, 1.0))[:, None]
        r = xs_ref[...] + jnp.dot(mean, w_ref[...],
                                  preferred_element_type=_f32)
        if act == "relu":
            o_ref[...] = jnp.maximum(r, 0.0)
        else:
            m = jnp.max(r, axis=1, keepdims=True)
            e = jnp.exp(r - m)
            o_ref[...] = e / jnp.sum(e, axis=1, keepdims=True)

    return pl.pallas_call(
        body,
        grid=(pl.cdiv(N_NODES, BR),),
        in_specs=[
            pl.BlockSpec((BR, dout), lambda i: (i, 0)),
            pl.BlockSpec((NC, BR, D_FEAT), lambda i: (0, i, 0)),
            pl.BlockSpec((NC * NS, BR), lambda i: (0, i)),
            pl.BlockSpec((D_FEAT, dout), lambda i: (0, 0)),
        ],
        out_specs=pl.BlockSpec((BR, dout), lambda i: (i, 0)),
        out_shape=jax.ShapeDtypeStruct((N_NODES, dout), _f32),
    )(xs, acc, cnt, w_neigh)


def kernel(x, edge_index, W1_self, W1_neigh, b1, W2_self, W2_neigh, b2):
    src = edge_index[0].astype(jnp.int32)
    dst = edge_index[1].astype(jnp.int32)
    dst3 = dst.reshape(NW, NCHUNK, CHUNK)

    acc1 = _sc_agg_k(x, src, dst3).reshape(NC, N_PAD, D_FEAT)
    cnt = _sc_counts_k(dst).reshape(NC * NS, N_PAD)
    xs = _tc_matmul(x, W1_self, b1)
    h = _tc_combine(xs, acc1, cnt, W1_neigh, "relu")

    acc2 = _sc_agg_k(h, src, dst3).reshape(NC, N_PAD, D_FEAT)
    hs = _tc_matmul(h, W2_self, b2)
    return _tc_combine(hs, acc2, cnt, W2_neigh, "softmax")
